# baseline (speedup 1.0000x reference)
import functools

import jax
import jax.numpy as jnp
from jax import lax
from jax.experimental import pallas as pl
from jax.experimental.pallas import tpu as pltpu
from jax.experimental.pallas import tpu_sc as plsc

NU = 10000
NM = 10000
D = 128
R = 5
E = 100000
N = R * E
K = 80
CH = N // K
NS = 16
TRIPS = (CH + NS - 1) // NS
ROWS_PER_TILE = 624


def _dense_body(rfeat_ref, pwu_ref, swu_ref, rwu_ref, pwm_ref, swm_ref, rwm_ref,
                rf0_ref, pa0_ref, rf1_ref, pa1_ref):
    x = rfeat_ref[0]
    for rw_ref, sw_ref, pw_ref, rf_ref, pa_ref in (
            (rwu_ref, swu_ref, pwu_ref, rf0_ref, pa0_ref),
            (rwm_ref, swm_ref, pwm_ref, rf1_ref, pa1_ref)):
        rw = rw_ref[0]
        rf = lax.dot_general(x, rw, (((1,), (1,)), ((), ())),
                             preferred_element_type=jnp.float32)
        sg = jax.nn.sigmoid(x @ sw_ref[0, 0])
        pa = jax.nn.sigmoid(x @ pw_ref[0, 0])
        rf_ref[0] = rf * sg[:, None]
        pa_ref[0] = jnp.broadcast_to(pa[:, None], pa.shape + (16,))


def _dense_phase(review_feat, prob_w_um, score_w_um, review_w_um,
                 prob_w_mu, score_w_mu, review_w_mu):
    be = 1000
    grid = (R, E // be)
    return pl.pallas_call(
        _dense_body,
        grid=grid,
        in_specs=[
            pl.BlockSpec((1, be, D), lambda r, b: (r, b, 0)),
            pl.BlockSpec((1, 1, D), lambda r, b: (r, 0, 0)),
            pl.BlockSpec((1, 1, D), lambda r, b: (r, 0, 0)),
            pl.BlockSpec((1, D, D), lambda r, b: (r, 0, 0)),
            pl.BlockSpec((1, 1, D), lambda r, b: (r, 0, 0)),
            pl.BlockSpec((1, 1, D), lambda r, b: (r, 0, 0)),
            pl.BlockSpec((1, D, D), lambda r, b: (r, 0, 0)),
        ],
        out_specs=[
            pl.BlockSpec((1, be, D), lambda r, b: (r, b, 0)),
            pl.BlockSpec((1, be, 16), lambda r, b: (r, b, 0)),
            pl.BlockSpec((1, be, D), lambda r, b: (r, b, 0)),
            pl.BlockSpec((1, be, 16), lambda r, b: (r, b, 0)),
        ],
        out_shape=[
            jax.ShapeDtypeStruct((R, E, D), jnp.float32),
            jax.ShapeDtypeStruct((R, E, 16), jnp.float32),
            jax.ShapeDtypeStruct((R, E, D), jnp.float32),
            jax.ShapeDtypeStruct((R, E, 16), jnp.float32),
        ],
    )(review_feat, prob_w_um[:, None, :], score_w_um[:, None, :], review_w_um,
      prob_w_mu[:, None, :], score_w_mu[:, None, :], review_w_mu)


def _wx_body(wu_ref, wm_ref, ucj_ref, mcj_ref, wx0_ref, wx1_ref):
    ucj = ucj_ref[...]
    mcj = mcj_ref[...]
    wx0_ref[0, :, :D] = wu_ref[0] * ucj
    wx0_ref[0, :, D:] = jnp.broadcast_to(ucj, (ucj.shape[0], D))
    wx1_ref[0, :, :D] = wm_ref[0] * mcj
    wx1_ref[0, :, D:] = jnp.broadcast_to(mcj, (mcj.shape[0], D))


def _wx_phase(W_user, W_movie, user_cj, movie_cj):
    bv = 1000
    return pl.pallas_call(
        _wx_body,
        grid=(R, NU // bv),
        in_specs=[
            pl.BlockSpec((1, bv, D), lambda r, b: (r, b, 0)),
            pl.BlockSpec((1, bv, D), lambda r, b: (r, b, 0)),
            pl.BlockSpec((bv, 1), lambda r, b: (b, 0)),
            pl.BlockSpec((bv, 1), lambda r, b: (b, 0)),
        ],
        out_specs=[
            pl.BlockSpec((1, bv, 2 * D), lambda r, b: (r, b, 0)),
            pl.BlockSpec((1, bv, 2 * D), lambda r, b: (r, b, 0)),
        ],
        out_shape=[
            jax.ShapeDtypeStruct((R, NU, 2 * D), jnp.float32),
            jax.ShapeDtypeStruct((R, NM, 2 * D), jnp.float32),
        ],
    )(W_user, W_movie, user_cj, movie_cj)


MC = 2 * K


def _sc_body(Wx0, Wx1,
             meta0, pa0, rf0, meta1, pa1, rf1,
             ufeat_out, ifeat_out,
             meta_a, pa_a, rf_a, meta_b, pa_b, rf_b,
             w_v, sidx_v, zb_v, acc,
             lsem_a, lsem_b, gsem, ssem):
    bufs = ((meta_a, pa_a, rf_a, lsem_a),
            (meta_b, pa_b, rf_b, lsem_b))
    c = lax.axis_index("c")
    s = lax.axis_index("s")

    def _zrow(i, carry):
        for l in range(8):
            zb_v[i, pl.ds(l * 16, 16)] = jnp.zeros((16,), jnp.float32)
        return carry
    lax.fori_loop(0, 16, _zrow, 0)
    ntrips = jnp.where(s == NS - 1, 40, 39)

    def _zcopy(j, carry):
        pltpu.sync_copy(zb_v, acc.at[pl.ds(s * ROWS_PER_TILE + j * 16, 16)])
        return carry
    lax.fori_loop(0, ntrips, _zcopy, 0)
    plsc.subcore_barrier()

    def _process(meta_hbm, pa_hbm, rf_hbm, wtab_hbm):
        def _linear_descs(buf, ci):
            meta_v, pa_v, rf_v, lsem = buf
            return lsem, (
                (meta_hbm.at[ci], meta_v),
                (pa_hbm.at[ci], pa_v),
                (rf_hbm.at[pl.ds(ci * K, K)], rf_v),
            )

        def _issue_linear(buf, ci):
            lsem, descs = _linear_descs(buf, ci)
            for src, dst in descs:
                pltpu.async_copy(src, dst, lsem)

        def _wait_linear(buf, ci):
            lsem, descs = _linear_descs(buf, ci)
            for src, dst in descs:
                pltpu.make_async_copy(src, dst, lsem).wait()

        def _gather_desc(buf):
            meta_v, pa_v, rf_v, lsem = buf
            return (wtab_hbm.at[meta_v.at[pl.ds(0, K)]], w_v, gsem)

        def _scatter_desc(buf):
            meta_v, pa_v, rf_v, lsem = buf
            return (rf_v, acc.at[sidx_v], ssem)

        def _load_sidx(meta_v, valid):
            dump = NU + lax.iota(jnp.int32, 16)
            for g in range(K // 16):
                sl = pl.ds(g * 16, 16)
                sidx_v[sl] = jnp.where(valid, meta_v[pl.ds(K + g * 16, 16)],
                                       dump)

        def _do_chunk(buf, ci):
            meta_v, pa_v, rf_v, lsem = buf
            nbuf = bufs[1] if buf is bufs[0] else bufs[0]
            nci_c = jnp.minimum(ci + NS, CH - 1)

            gs, gd, gsm = _scatter_desc(nbuf)
            pltpu.make_async_copy(gs, gd, gsm).wait()
            _issue_linear(nbuf, nci_c)

            ws, wd, wsm = _gather_desc(buf)
            pltpu.make_async_copy(ws, wd, wsm).wait()

            def _eblock(eb, carry2):
                e0 = eb * 8
                for de in range(8):
                    e = e0 + de
                    pa_s = pa_v[pl.ds(e * 16, 16)]
                    cj_s = w_v[e, pl.ds(D, 16)]
                    for l in range(8):
                        sl = pl.ds(l * 16, 16)
                        rf_v[e, sl] = (w_v[e, sl] * pa_s
                                       + rf_v[e, sl] * cj_s)
                return carry2
            lax.fori_loop(0, K // 8, _eblock, 0)

            _load_sidx(meta_v, ci < CH)
            _wait_linear(nbuf, nci_c)
            ns, nd, nsm = _gather_desc(nbuf)
            pltpu.async_copy(ns, nd, nsm)

            ss, sd, ssm = _scatter_desc(buf)
            pltpu.async_copy(ss, sd, ssm, add=True)

        _issue_linear(bufs[0], s)
        _wait_linear(bufs[0], s)
        gs0, gd0, gsm0 = _gather_desc(bufs[0])
        pltpu.async_copy(gs0, gd0, gsm0)
        _load_sidx(bufs[0][0], jnp.bool_(False))
        ps, pd, psm = _scatter_desc(bufs[1])
        pltpu.async_copy(ps, pd, psm, add=True)

        def _pair_trip(m, carry):
            _do_chunk(bufs[0], s + (2 * m) * NS)
            _do_chunk(bufs[1], s + (2 * m + 1) * NS)
            return carry
        lax.fori_loop(0, (TRIPS + 1) // 2, _pair_trip, 0)
        ds_, dd_, dsm_ = _gather_desc(bufs[0])
        pltpu.make_async_copy(ds_, dd_, dsm_).wait()
        fs, fd, fsm = _scatter_desc(bufs[1])
        pltpu.make_async_copy(fs, fd, fsm).wait()

    @pl.when(c == 0)
    def _():
        _process(meta0, pa0, rf0, Wx0)

    @pl.when(c == 1)
    def _():
        _process(meta1, pa1, rf1, Wx1)

    plsc.subcore_barrier()

    @pl.when(c == 0)
    def _():
        def _fcopy(j, carry):
            off = s * ROWS_PER_TILE + j * 16
            pltpu.sync_copy(acc.at[pl.ds(off, 16)],
                            ifeat_out.at[pl.ds(off, 16)])
            return carry
        lax.fori_loop(0, ntrips, _fcopy, 0)

    @pl.when(c == 1)
    def _():
        def _fcopy(j, carry):
            off = s * ROWS_PER_TILE + j * 16
            pltpu.sync_copy(acc.at[pl.ds(off, 16)],
                            ufeat_out.at[pl.ds(off, 16)])
            return carry
        lax.fori_loop(0, ntrips, _fcopy, 0)


def _sparse_phase(Wx0, Wx1, d0, d1):
    mesh = plsc.VectorSubcoreMesh(core_axis_name="c", subcore_axis_name="s")
    fn = pl.kernel(
        _sc_body,
        out_type=(jax.ShapeDtypeStruct((NU, D), jnp.float32),
                  jax.ShapeDtypeStruct((NM, D), jnp.float32)),
        mesh=mesh,
        scratch_types=(
            [pltpu.VMEM((MC,), jnp.int32)]
            + [pltpu.VMEM((16 * K,), jnp.float32)]
            + [pltpu.VMEM((K, D), jnp.float32)]
        ) * 2 + [
            pltpu.VMEM((K, 2 * D), jnp.float32),
            pltpu.VMEM((K,), jnp.int32),
            pltpu.VMEM((16, D), jnp.float32),
            pltpu.VMEM_SHARED((NU + 16, D), jnp.float32),
        ] + [pltpu.SemaphoreType.DMA] * 4,
    )
    return fn(Wx0, Wx1, *d0, *d1)


def _gelu_exact(x):
    return x * 0.5 * (1.0 + lax.erf(x * 0.7071067811865476))


def _tail_body(uf_ref, if_ref, uci_ref, ici_ref, uW_ref, ub_ref, iW_ref, ib_ref,
               uo_ref, io_ref):
    uf = _gelu_exact(uf_ref[...] * uci_ref[...])
    io = _gelu_exact(if_ref[...] * ici_ref[...])
    uo_ref[...] = uf @ uW_ref[...].T + ub_ref[...][None, :]
    io_ref[...] = io @ iW_ref[...].T + ib_ref[...][None, :]


def _tail_phase(ufeat, ifeat, user_ci, movie_ci, ufc_W, ufc_b, ifc_W, ifc_b):
    grid = 10
    blk_u = NU // grid
    blk_m = NM // grid
    return pl.pallas_call(
        _tail_body,
        grid=(grid,),
        in_specs=[
            pl.BlockSpec((blk_u, D), lambda i: (i, 0)),
            pl.BlockSpec((blk_m, D), lambda i: (i, 0)),
            pl.BlockSpec((blk_u, 1), lambda i: (i, 0)),
            pl.BlockSpec((blk_m, 1), lambda i: (i, 0)),
            pl.BlockSpec((D, D), lambda i: (0, 0)),
            pl.BlockSpec((D,), lambda i: (0,)),
            pl.BlockSpec((D, D), lambda i: (0, 0)),
            pl.BlockSpec((D,), lambda i: (0,)),
        ],
        out_specs=[
            pl.BlockSpec((blk_u, D), lambda i: (i, 0)),
            pl.BlockSpec((blk_m, D), lambda i: (i, 0)),
        ],
        out_shape=[
            jax.ShapeDtypeStruct((NU, D), jnp.float32),
            jax.ShapeDtypeStruct((NM, D), jnp.float32),
        ],
    )(ufeat, ifeat, user_ci, movie_ci, ufc_W, ufc_b, ifc_W, ifc_b)


def kernel(edge_index, review_feat, user_cj, user_ci, movie_cj, movie_ci,
           W_user, W_movie, prob_w_um, score_w_um, review_w_um,
           prob_w_mu, score_w_mu, review_w_mu, ufc_W, ufc_b, ifc_W, ifc_b):
    rf0, pa0, rf1, pa1 = _dense_phase(
        review_feat, prob_w_um, score_w_um, review_w_um,
        prob_w_mu, score_w_mu, review_w_mu)

    Wx0, Wx1 = _wx_phase(W_user, W_movie, user_cj, movie_cj)

    src = edge_index[:, 0, :].astype(jnp.int32)
    dst = edge_index[:, 1, :].astype(jnp.int32)
    roffs = (jnp.arange(R, dtype=jnp.int32) * NU)[:, None]

    def _meta(widx, sidx):
        return jnp.concatenate(
            [widx.reshape(CH, K), sidx.reshape(CH, K)], axis=1)

    d0 = (_meta((src + roffs).reshape(N), dst.reshape(N)),
          pa0.reshape(CH, 16 * K), rf0.reshape(N, D))
    d1 = (_meta((dst + roffs).reshape(N), src.reshape(N)),
          pa1.reshape(CH, 16 * K), rf1.reshape(N, D))

    ufeat, ifeat = _sparse_phase(
        Wx0.reshape(R * NU, 2 * D), Wx1.reshape(R * NM, 2 * D), d0, d1)

    return _tail_phase(ufeat, ifeat, user_ci, movie_ci,
                       ufc_W, ufc_b, ifc_W, ifc_b)
